# Initial kernel scaffold; baseline (speedup 1.0000x reference)
import functools

import jax
import jax.numpy as jnp
from jax import lax
from jax.experimental import pallas as pl
from jax.experimental.pallas import tpu as pltpu
from jax.experimental.pallas import tpu_sc as plsc

N_USER = 20000
N_ITEM = 30000
N_NODES = N_USER + N_ITEM
EMB = 64
HALF = 32
N_EDGES = 800000
N_TILES = 16
ROWS_PT = N_NODES // N_TILES
CHUNK = 128
CHUNKS_PER_BLOCK = 56
BLOCK_E = CHUNK * CHUNKS_PER_BLOCK
BLOCKS = 7
EDGES_PT = BLOCK_E * BLOCKS
E_PAD = EDGES_PT * N_TILES
WB_ROWS = 125
WB_CHUNKS = ROWS_PT // WB_ROWS
LANES = 16


def _sc_body(ego0, srcr, dstr, wr, out_sum, ego_scr,
             spmem, src_blk, w_blk, dst_blk, gbuf, zbuf, ebuf, abuf, gsem):
    c = lax.axis_index("c")
    s = lax.axis_index("s")
    col_off = c * N_NODES
    offv = jnp.broadcast_to(col_off, (LANES,)).astype(jnp.int32)
    third = jnp.float32(1.0 / 3.0)

    zv = jnp.zeros((LANES,), jnp.float32)

    def zfill(i, _):
        for h in range(2):
            zbuf[i, pl.ds(h * LANES, LANES)] = zv
        return 0

    lax.fori_loop(0, WB_ROWS, zfill, 0)

    def layer(src_ref, first, last):
        for k in range(WB_CHUNKS):
            pltpu.sync_copy(zbuf,
                            spmem.at[pl.ds(s * ROWS_PT + k * WB_ROWS, WB_ROWS)])
        plsc.subcore_barrier()

        def block_body(b, _):
            eoff = s * EDGES_PT + b * BLOCK_E
            pltpu.sync_copy(srcr.at[pl.ds(eoff, BLOCK_E)], src_blk)
            pltpu.sync_copy(wr.at[pl.ds(eoff, BLOCK_E)], w_blk)
            pltpu.sync_copy(
                dstr.at[s, pl.ds(b * CHUNKS_PER_BLOCK, CHUNKS_PER_BLOCK)],
                dst_blk)

            def add_off(i, _):
                src_blk[pl.ds(i * LANES, LANES)] = (
                    src_blk[pl.ds(i * LANES, LANES)] + offv)
                return 0

            lax.fori_loop(0, BLOCK_E // LANES, add_off, 0)

            def chunk_body(j, _):
                pltpu.async_copy(
                    src_ref.at[src_blk.at[pl.ds(j * CHUNK, CHUNK)]],
                    gbuf.at[0], gsem).wait()
                for i in range(CHUNK):
                    wv = jnp.broadcast_to(w_blk[j * CHUNK + i], (LANES,))
                    for h in range(2):
                        gbuf[0, i, pl.ds(h * LANES, LANES)] = (
                            gbuf[0, i, pl.ds(h * LANES, LANES)] * wv)
                pltpu.sync_copy(gbuf, spmem.at[dst_blk.at[j]], add=True)
                return 0

            lax.fori_loop(0, CHUNKS_PER_BLOCK, chunk_body, 0)
            return 0

        lax.fori_loop(0, BLOCKS, block_body, 0)
        plsc.subcore_barrier()

        def wb_body(k, _):
            r0 = s * ROWS_PT + k * WB_ROWS
            pltpu.sync_copy(spmem.at[pl.ds(r0, WB_ROWS)], ebuf)
            if not last:
                pltpu.sync_copy(ebuf, ego_scr.at[pl.ds(col_off + r0, WB_ROWS)])
            if not first:
                pltpu.sync_copy(out_sum.at[pl.ds(col_off + r0, WB_ROWS)], abuf)

            def wb_row(i, _):
                for h in range(2):
                    v = ebuf[i, pl.ds(h * LANES, LANES)] * third
                    if first:
                        abuf[i, pl.ds(h * LANES, LANES)] = v
                    else:
                        abuf[i, pl.ds(h * LANES, LANES)] = (
                            abuf[i, pl.ds(h * LANES, LANES)] + v)
                return 0

            lax.fori_loop(0, WB_ROWS, wb_row, 0)
            pltpu.sync_copy(abuf, out_sum.at[pl.ds(col_off + r0, WB_ROWS)])
            return 0

        lax.fori_loop(0, WB_CHUNKS, wb_body, 0)

    layer(ego0, first=True, last=False)
    plsc.subcore_barrier()
    layer(ego_scr, first=False, last=False)
    plsc.subcore_barrier()
    layer(ego_scr, first=False, last=True)


_sc_kernel = functools.partial(
    pl.kernel,
    out_type=(
        jax.ShapeDtypeStruct((2 * N_NODES, HALF), jnp.float32),
        jax.ShapeDtypeStruct((2 * N_NODES, HALF), jnp.float32),
    ),
    mesh=plsc.VectorSubcoreMesh(core_axis_name="c", subcore_axis_name="s"),
    scratch_types=[
        pltpu.VMEM_SHARED((N_NODES, HALF), jnp.float32),
        pltpu.VMEM((BLOCK_E,), jnp.int32),
        pltpu.VMEM((BLOCK_E,), jnp.float32),
        pltpu.VMEM((CHUNKS_PER_BLOCK, 1, CHUNK), jnp.int32),
        pltpu.VMEM((1, CHUNK, HALF), jnp.float32),
        pltpu.VMEM((WB_ROWS, HALF), jnp.float32),
        pltpu.VMEM((WB_ROWS, HALF), jnp.float32),
        pltpu.VMEM((WB_ROWS, HALF), jnp.float32),
        pltpu.SemaphoreType.DMA,
    ],
)(_sc_body)


def kernel(user_emb, item_emb, edge_index, edge_weight):
    ego0 = jnp.concatenate([user_emb, item_emb], axis=0)
    ego0 = ego0.reshape(N_NODES, 2, HALF).transpose(1, 0, 2).reshape(
        2 * N_NODES, HALF)
    pad = E_PAD - N_EDGES
    src = jnp.concatenate([edge_index[0], jnp.zeros((pad,), jnp.int32)])
    dst = jnp.concatenate([edge_index[1], jnp.zeros((pad,), jnp.int32)])
    dst = dst.reshape(N_TILES, BLOCKS * CHUNKS_PER_BLOCK, 1, CHUNK)
    w = jnp.concatenate([edge_weight, jnp.zeros((pad,), jnp.float32)])
    out_sum, _ = _sc_kernel(ego0, src, dst, w)
    out = out_sum.reshape(2, N_NODES, HALF).transpose(1, 0, 2).reshape(
        N_NODES, EMB)
    return out[:N_USER], out[N_USER:]


# speedup vs baseline: 5.4755x; 5.4755x over previous
import functools

import jax
import jax.numpy as jnp
from jax import lax
from jax.experimental import pallas as pl
from jax.experimental.pallas import tpu as pltpu
from jax.experimental.pallas import tpu_sc as plsc

N_USER = 20000
N_ITEM = 30000
N_NODES = N_USER + N_ITEM
N_PAD = 50048
EMB = 64
HALF = 32
N_EDGES = 800000
N_TILES = 16
ROWS_PT = N_PAD // N_TILES
CHUNK = 128
CHUNKS_PER_BLOCK = 28
BLOCK_E = CHUNK * CHUNKS_PER_BLOCK
BLOCKS = 14
EDGES_PT = BLOCK_E * BLOCKS
E_PAD = EDGES_PT * N_TILES
WB_ROWS = 136
WB_CHUNKS = ROWS_PT // WB_ROWS
LANES = 16


def _sc_body(ego0, srcr, dstr, wr, out_sum, ego_scr,
             spmem, src_blk, w_blk, dst_blk, gbuf, ebuf, abuf, gsem):
    c = lax.axis_index("c")
    s = lax.axis_index("s")
    col_off = c * N_PAD
    offv = jnp.broadcast_to(col_off, (LANES,)).astype(jnp.int32)
    third = jnp.float32(1.0 / 3.0)

    zv = jnp.zeros((LANES,), jnp.float32)

    def layer(src_ref, first, last):
        def zfill(i, _):
            for h in range(2):
                ebuf[i, pl.ds(h * LANES, LANES)] = zv
            return 0

        lax.fori_loop(0, WB_ROWS, zfill, 0)
        for k in range(WB_CHUNKS):
            pltpu.sync_copy(ebuf,
                            spmem.at[pl.ds(s * ROWS_PT + k * WB_ROWS, WB_ROWS)])
        plsc.subcore_barrier()

        def block_body(b, _):
            eoff = s * EDGES_PT + b * BLOCK_E
            pltpu.sync_copy(srcr.at[pl.ds(eoff, BLOCK_E)], src_blk)
            pltpu.sync_copy(wr.at[pl.ds(eoff, BLOCK_E)], w_blk)
            pltpu.sync_copy(
                dstr.at[s, pl.ds(b * CHUNKS_PER_BLOCK, CHUNKS_PER_BLOCK)],
                dst_blk)

            def add_off(i, _):
                src_blk[pl.ds(i * LANES, LANES)] = (
                    src_blk[pl.ds(i * LANES, LANES)] + offv)
                return 0

            lax.fori_loop(0, BLOCK_E // LANES, add_off, 0)

            def chunk_body(j, _):
                pltpu.async_copy(
                    src_ref.at[src_blk.at[pl.ds(j * CHUNK, CHUNK)]],
                    gbuf, gsem).wait()
                for g in range(CHUNK // LANES):
                    wvec = w_blk[pl.ds(j * CHUNK + g * LANES, LANES)]
                    for i in range(LANES):
                        row = g * LANES + i
                        wv = jnp.broadcast_to(wvec[i], (LANES,))
                        for h in range(2):
                            gbuf[row, pl.ds(h * LANES, LANES)] = (
                                gbuf[row, pl.ds(h * LANES, LANES)] * wv)
                pltpu.sync_copy(gbuf, spmem.at[dst_blk.at[j]], add=True)
                return 0

            lax.fori_loop(0, CHUNKS_PER_BLOCK, chunk_body, 0)
            return 0

        lax.fori_loop(0, BLOCKS, block_body, 0)
        plsc.subcore_barrier()

        def wb_body(k, _):
            r0 = s * ROWS_PT + k * WB_ROWS
            pltpu.sync_copy(spmem.at[pl.ds(r0, WB_ROWS)], ebuf)
            if not last:
                pltpu.sync_copy(ebuf, ego_scr.at[pl.ds(col_off + r0, WB_ROWS)])
            if not first:
                pltpu.sync_copy(out_sum.at[pl.ds(col_off + r0, WB_ROWS)], abuf)

            def wb_row(i, _):
                for h in range(2):
                    v = ebuf[i, pl.ds(h * LANES, LANES)] * third
                    if first:
                        abuf[i, pl.ds(h * LANES, LANES)] = v
                    else:
                        abuf[i, pl.ds(h * LANES, LANES)] = (
                            abuf[i, pl.ds(h * LANES, LANES)] + v)
                return 0

            lax.fori_loop(0, WB_ROWS, wb_row, 0)
            pltpu.sync_copy(abuf, out_sum.at[pl.ds(col_off + r0, WB_ROWS)])
            return 0

        lax.fori_loop(0, WB_CHUNKS, wb_body, 0)

    layer(ego0, first=True, last=False)
    plsc.subcore_barrier()
    layer(ego_scr, first=False, last=False)
    plsc.subcore_barrier()
    layer(ego_scr, first=False, last=True)


_sc_kernel = functools.partial(
    pl.kernel,
    out_type=(
        jax.ShapeDtypeStruct((2 * N_PAD, HALF), jnp.float32),
        jax.ShapeDtypeStruct((2 * N_PAD, HALF), jnp.float32),
    ),
    mesh=plsc.VectorSubcoreMesh(core_axis_name="c", subcore_axis_name="s"),
    compiler_params=pltpu.CompilerParams(use_tc_tiling_on_sc=False),
    scratch_types=[
        pltpu.VMEM_SHARED((N_PAD, HALF), jnp.float32),
        pltpu.VMEM((BLOCK_E,), jnp.int32),
        pltpu.VMEM((BLOCK_E,), jnp.float32),
        pltpu.VMEM((CHUNKS_PER_BLOCK, CHUNK), jnp.int32),
        pltpu.VMEM((CHUNK, HALF), jnp.float32),
        pltpu.VMEM((WB_ROWS, HALF), jnp.float32),
        pltpu.VMEM((WB_ROWS, HALF), jnp.float32),
        pltpu.SemaphoreType.DMA,
    ],
)(_sc_body)


def kernel(user_emb, item_emb, edge_index, edge_weight):
    ego0 = jnp.concatenate([user_emb, item_emb], axis=0)
    ego0 = ego0.reshape(N_NODES, 2, HALF).transpose(1, 0, 2)
    ego0 = jnp.pad(ego0, ((0, 0), (0, N_PAD - N_NODES), (0, 0)))
    ego0 = ego0.reshape(2 * N_PAD, HALF)
    pad = E_PAD - N_EDGES
    src = jnp.concatenate([edge_index[0], jnp.zeros((pad,), jnp.int32)])
    dst = jnp.concatenate([edge_index[1], jnp.zeros((pad,), jnp.int32)])
    dst = dst.reshape(N_TILES, BLOCKS * CHUNKS_PER_BLOCK, CHUNK)
    w = jnp.concatenate([edge_weight, jnp.zeros((pad,), jnp.float32)])
    out_sum, _ = _sc_kernel(ego0, src, dst, w)
    out = out_sum.reshape(2, N_PAD, HALF)[:, :N_NODES].transpose(
        1, 0, 2).reshape(N_NODES, EMB)
    return out[:N_USER], out[N_USER:]


# speedup vs baseline: 8.2375x; 1.5044x over previous
import functools

import jax
import jax.numpy as jnp
from jax import lax
from jax.experimental import pallas as pl
from jax.experimental.pallas import tpu as pltpu
from jax.experimental.pallas import tpu_sc as plsc

N_USER = 20000
N_ITEM = 30000
N_NODES = N_USER + N_ITEM
N_PAD = 50048
EMB = 64
HALF = 32
N_EDGES = 800000
N_TILES = 16
ROWS_PT = N_PAD // N_TILES
CHUNK = 128
CHUNKS_PER_BLOCK = 28
BLOCK_E = CHUNK * CHUNKS_PER_BLOCK
BLOCKS = 14
EDGES_PT = BLOCK_E * BLOCKS
E_PAD = EDGES_PT * N_TILES
WB_ROWS = 136
WB_CHUNKS = ROWS_PT // WB_ROWS
LANES = 16


def _sc_body(ego0, srcr, dstr, wr, out_sum, ego_scr,
             spmem, src_blk, w_blk, dst_blk, gbufa, gbufb, ebuf, abuf,
             gsema, gsemb, ssema, ssemb):
    c = lax.axis_index("c")
    s = lax.axis_index("s")
    col_off = c * N_PAD
    offv = jnp.broadcast_to(col_off, (LANES,)).astype(jnp.int32)
    third = jnp.float32(1.0 / 3.0)

    zv = jnp.zeros((LANES,), jnp.float32)

    def layer(src_ref, first, last):
        def zfill(i, _):
            for h in range(2):
                ebuf[i, pl.ds(h * LANES, LANES)] = zv
            return 0

        lax.fori_loop(0, WB_ROWS, zfill, 0)
        for k in range(WB_CHUNKS):
            pltpu.sync_copy(ebuf,
                            spmem.at[pl.ds(s * ROWS_PT + k * WB_ROWS, WB_ROWS)])
        plsc.subcore_barrier()

        def block_body(b, _):
            eoff = s * EDGES_PT + b * BLOCK_E
            pltpu.sync_copy(srcr.at[pl.ds(eoff, BLOCK_E)], src_blk)
            pltpu.sync_copy(wr.at[pl.ds(eoff, BLOCK_E)], w_blk)
            pltpu.sync_copy(
                dstr.at[s, pl.ds(b * CHUNKS_PER_BLOCK, CHUNKS_PER_BLOCK)],
                dst_blk)

            def add_off(i, _):
                src_blk[pl.ds(i * LANES, LANES)] = (
                    src_blk[pl.ds(i * LANES, LANES)] + offv)
                return 0

            lax.fori_loop(0, BLOCK_E // LANES, add_off, 0)

            gbufs = [gbufa, gbufb]
            gsems = [gsema, gsemb]
            ssems = [ssema, ssemb]

            def gidx(j):
                return src_blk.at[pl.ds(j * CHUNK, CHUNK)]

            for p in range(2):
                pltpu.async_copy(src_ref.at[gidx(p)], gbufs[p], gsems[p])

            def pair_body(j2, _):
                for p in range(2):
                    j = 2 * j2 + p
                    pltpu.make_async_copy(
                        src_ref.at[gidx(j)], gbufs[p], gsems[p]).wait()
                    for g in range(CHUNK // LANES):
                        wvec = w_blk[pl.ds(j * CHUNK + g * LANES, LANES)]
                        for i in range(LANES):
                            row = g * LANES + i
                            wv = jnp.broadcast_to(wvec[i], (LANES,))
                            for h in range(2):
                                gbufs[p][row, pl.ds(h * LANES, LANES)] = (
                                    gbufs[p][row, pl.ds(h * LANES, LANES)]
                                    * wv)
                    pltpu.async_copy(gbufs[p], spmem.at[dst_blk.at[j]],
                                     ssems[p], add=True)

                    @pl.when(j + 2 < CHUNKS_PER_BLOCK)
                    def _():
                        pltpu.make_async_copy(
                            gbufs[p], spmem.at[dst_blk.at[j]],
                            ssems[p]).wait()
                        pltpu.async_copy(src_ref.at[gidx(j + 2)], gbufs[p],
                                         gsems[p])
                return 0

            lax.fori_loop(0, CHUNKS_PER_BLOCK // 2, pair_body, 0)
            for p in range(2):
                j = CHUNKS_PER_BLOCK - 2 + p
                pltpu.make_async_copy(
                    gbufs[p], spmem.at[dst_blk.at[j]], ssems[p]).wait()
            return 0

        lax.fori_loop(0, BLOCKS, block_body, 0)
        plsc.subcore_barrier()

        def wb_body(k, _):
            r0 = s * ROWS_PT + k * WB_ROWS
            pltpu.sync_copy(spmem.at[pl.ds(r0, WB_ROWS)], ebuf)
            if not last:
                pltpu.sync_copy(ebuf, ego_scr.at[pl.ds(col_off + r0, WB_ROWS)])
            if not first:
                pltpu.sync_copy(out_sum.at[pl.ds(col_off + r0, WB_ROWS)], abuf)

            def wb_row(i, _):
                for h in range(2):
                    v = ebuf[i, pl.ds(h * LANES, LANES)] * third
                    if first:
                        abuf[i, pl.ds(h * LANES, LANES)] = v
                    else:
                        abuf[i, pl.ds(h * LANES, LANES)] = (
                            abuf[i, pl.ds(h * LANES, LANES)] + v)
                return 0

            lax.fori_loop(0, WB_ROWS, wb_row, 0)
            pltpu.sync_copy(abuf, out_sum.at[pl.ds(col_off + r0, WB_ROWS)])
            return 0

        lax.fori_loop(0, WB_CHUNKS, wb_body, 0)

    layer(ego0, first=True, last=False)
    plsc.subcore_barrier()
    layer(ego_scr, first=False, last=False)
    plsc.subcore_barrier()
    layer(ego_scr, first=False, last=True)


_sc_kernel = functools.partial(
    pl.kernel,
    out_type=(
        jax.ShapeDtypeStruct((2 * N_PAD, HALF), jnp.float32),
        jax.ShapeDtypeStruct((2 * N_PAD, HALF), jnp.float32),
    ),
    mesh=plsc.VectorSubcoreMesh(core_axis_name="c", subcore_axis_name="s"),
    compiler_params=pltpu.CompilerParams(use_tc_tiling_on_sc=False),
    scratch_types=[
        pltpu.VMEM_SHARED((N_PAD, HALF), jnp.float32),
        pltpu.VMEM((BLOCK_E,), jnp.int32),
        pltpu.VMEM((BLOCK_E,), jnp.float32),
        pltpu.VMEM((CHUNKS_PER_BLOCK, CHUNK), jnp.int32),
        pltpu.VMEM((CHUNK, HALF), jnp.float32),
        pltpu.VMEM((CHUNK, HALF), jnp.float32),
        pltpu.VMEM((WB_ROWS, HALF), jnp.float32),
        pltpu.VMEM((WB_ROWS, HALF), jnp.float32),
        pltpu.SemaphoreType.DMA,
        pltpu.SemaphoreType.DMA,
        pltpu.SemaphoreType.DMA,
        pltpu.SemaphoreType.DMA,
    ],
)(_sc_body)


def kernel(user_emb, item_emb, edge_index, edge_weight):
    ego0 = jnp.concatenate([user_emb, item_emb], axis=0)
    ego0 = ego0.reshape(N_NODES, 2, HALF).transpose(1, 0, 2)
    ego0 = jnp.pad(ego0, ((0, 0), (0, N_PAD - N_NODES), (0, 0)))
    ego0 = ego0.reshape(2 * N_PAD, HALF)
    pad = E_PAD - N_EDGES
    src = jnp.concatenate([edge_index[0], jnp.zeros((pad,), jnp.int32)])
    dst = jnp.concatenate([edge_index[1], jnp.zeros((pad,), jnp.int32)])
    dst = dst.reshape(N_TILES, BLOCKS * CHUNKS_PER_BLOCK, CHUNK)
    w = jnp.concatenate([edge_weight, jnp.zeros((pad,), jnp.float32)])
    out_sum, _ = _sc_kernel(ego0, src, dst, w)
    out = out_sum.reshape(2, N_PAD, HALF)[:, :N_NODES].transpose(
        1, 0, 2).reshape(N_NODES, EMB)
    return out[:N_USER], out[N_USER:]


# speedup vs baseline: 9.5203x; 1.1557x over previous
import functools

import jax
import jax.numpy as jnp
from jax import lax
from jax.experimental import pallas as pl
from jax.experimental.pallas import tpu as pltpu
from jax.experimental.pallas import tpu_sc as plsc

N_USER = 20000
N_ITEM = 30000
N_NODES = N_USER + N_ITEM
N_PAD = 50048
EMB = 64
HALF = 32
N_EDGES = 800000
N_TILES = 16
ROWS_PT = N_PAD // N_TILES
CHUNK = 128
NBUF = 4
CHUNKS_PER_BLOCK = 28
BLOCK_E = CHUNK * CHUNKS_PER_BLOCK
BLOCKS = 14
EDGES_PT = BLOCK_E * BLOCKS
E_PAD = EDGES_PT * N_TILES
WB_FULL = ROWS_PT // CHUNK
WB_TAIL = ROWS_PT - WB_FULL * CHUNK
LANES = 16


def _sc_body(ego0, srcr, dstr, wr, out_sum, ego_scr, spmem,
             src_blk, w_blk, dst_blk, gb0, gb1, gb2, gb3,
             gs0, gs1, gs2, gs3, ss0, ss1, ss2, ss3):
    c = lax.axis_index("c")
    s = lax.axis_index("s")
    col_off = c * N_PAD
    offv = jnp.broadcast_to(col_off, (LANES,)).astype(jnp.int32)
    third = jnp.float32(1.0 / 3.0)
    zv = jnp.zeros((LANES,), jnp.float32)
    gbufs = [gb0, gb1, gb2, gb3]
    gsems = [gs0, gs1, gs2, gs3]
    ssems = [ss0, ss1, ss2, ss3]
    ebuf, abuf = gb0, gb1

    def layer(src_ref, first, last):
        def zfill(i, _):
            for h in range(2):
                ebuf[i, pl.ds(h * LANES, LANES)] = zv
            return 0

        lax.fori_loop(0, CHUNK, zfill, 0)

        def zcopy(k, _):
            pltpu.sync_copy(ebuf,
                            spmem.at[pl.ds(s * ROWS_PT + k * CHUNK, CHUNK)])
            return 0

        lax.fori_loop(0, WB_FULL, zcopy, 0)
        pltpu.sync_copy(
            ebuf.at[pl.ds(0, WB_TAIL)],
            spmem.at[pl.ds(s * ROWS_PT + WB_FULL * CHUNK, WB_TAIL)])
        plsc.subcore_barrier()

        def gidx(j):
            return src_blk.at[pl.ds(j * CHUNK, CHUNK)]

        def block_body(b, _):
            eoff = s * EDGES_PT + b * BLOCK_E
            pltpu.sync_copy(srcr.at[pl.ds(eoff, BLOCK_E)], src_blk)
            pltpu.sync_copy(wr.at[pl.ds(eoff, BLOCK_E)], w_blk)
            pltpu.sync_copy(
                dstr.at[s, pl.ds(b * CHUNKS_PER_BLOCK, CHUNKS_PER_BLOCK)],
                dst_blk)

            def add_off(i, _):
                src_blk[pl.ds(i * LANES, LANES)] = (
                    src_blk[pl.ds(i * LANES, LANES)] + offv)
                return 0

            lax.fori_loop(0, BLOCK_E // LANES, add_off, 0)

            for p in range(NBUF):
                pltpu.async_copy(src_ref.at[gidx(p)], gbufs[p], gsems[p])

            def quad_body(jq, _):
                for p in range(NBUF):
                    j = NBUF * jq + p
                    pltpu.make_async_copy(
                        src_ref.at[gidx(j)], gbufs[p], gsems[p]).wait()
                    for g in range(CHUNK // LANES):
                        wvec = w_blk[pl.ds(j * CHUNK + g * LANES, LANES)]
                        for i in range(LANES):
                            row = g * LANES + i
                            wv = jnp.broadcast_to(wvec[i], (LANES,))
                            for h in range(2):
                                gbufs[p][row, pl.ds(h * LANES, LANES)] = (
                                    gbufs[p][row, pl.ds(h * LANES, LANES)]
                                    * wv)
                    pltpu.async_copy(gbufs[p], spmem.at[dst_blk.at[j]],
                                     ssems[p], add=True)

                    @pl.when(j + NBUF < CHUNKS_PER_BLOCK)
                    def _():
                        pltpu.make_async_copy(
                            gbufs[p], spmem.at[dst_blk.at[j]],
                            ssems[p]).wait()
                        pltpu.async_copy(src_ref.at[gidx(j + NBUF)],
                                         gbufs[p], gsems[p])
                return 0

            lax.fori_loop(0, CHUNKS_PER_BLOCK // NBUF, quad_body, 0)
            for p in range(NBUF):
                j = CHUNKS_PER_BLOCK - NBUF + p
                pltpu.make_async_copy(
                    gbufs[p], spmem.at[dst_blk.at[j]], ssems[p]).wait()
            return 0

        lax.fori_loop(0, BLOCKS, block_body, 0)
        plsc.subcore_barrier()

        def wb_chunk(r0, rows):
            pltpu.sync_copy(spmem.at[pl.ds(r0, rows)],
                            ebuf.at[pl.ds(0, rows)])
            if not last:
                pltpu.sync_copy(ebuf.at[pl.ds(0, rows)],
                                ego_scr.at[pl.ds(col_off + r0, rows)])
            if not first:
                pltpu.sync_copy(out_sum.at[pl.ds(col_off + r0, rows)],
                                abuf.at[pl.ds(0, rows)])

            def wb_row(i, _):
                for h in range(2):
                    v = ebuf[i, pl.ds(h * LANES, LANES)] * third
                    if first:
                        abuf[i, pl.ds(h * LANES, LANES)] = v
                    else:
                        abuf[i, pl.ds(h * LANES, LANES)] = (
                            abuf[i, pl.ds(h * LANES, LANES)] + v)
                return 0

            lax.fori_loop(0, rows, wb_row, 0)
            pltpu.sync_copy(abuf.at[pl.ds(0, rows)],
                            out_sum.at[pl.ds(col_off + r0, rows)])

        def wb_body(k, _):
            wb_chunk(s * ROWS_PT + k * CHUNK, CHUNK)
            return 0

        lax.fori_loop(0, WB_FULL, wb_body, 0)
        wb_chunk(s * ROWS_PT + WB_FULL * CHUNK, WB_TAIL)

    layer(ego0, first=True, last=False)
    plsc.subcore_barrier()
    layer(ego_scr, first=False, last=False)
    plsc.subcore_barrier()
    layer(ego_scr, first=False, last=True)


_sc_kernel = functools.partial(
    pl.kernel,
    out_type=(
        jax.ShapeDtypeStruct((2 * N_PAD, HALF), jnp.float32),
        jax.ShapeDtypeStruct((2 * N_PAD, HALF), jnp.float32),
    ),
    mesh=plsc.VectorSubcoreMesh(core_axis_name="c", subcore_axis_name="s"),
    compiler_params=pltpu.CompilerParams(use_tc_tiling_on_sc=False),
    scratch_types=[
        pltpu.VMEM_SHARED((N_PAD, HALF), jnp.float32),
        pltpu.VMEM((BLOCK_E,), jnp.int32),
        pltpu.VMEM((BLOCK_E,), jnp.float32),
        pltpu.VMEM((CHUNKS_PER_BLOCK, CHUNK), jnp.int32),
        pltpu.VMEM((CHUNK, HALF), jnp.float32),
        pltpu.VMEM((CHUNK, HALF), jnp.float32),
        pltpu.VMEM((CHUNK, HALF), jnp.float32),
        pltpu.VMEM((CHUNK, HALF), jnp.float32),
        pltpu.SemaphoreType.DMA,
        pltpu.SemaphoreType.DMA,
        pltpu.SemaphoreType.DMA,
        pltpu.SemaphoreType.DMA,
        pltpu.SemaphoreType.DMA,
        pltpu.SemaphoreType.DMA,
        pltpu.SemaphoreType.DMA,
        pltpu.SemaphoreType.DMA,
    ],
)(_sc_body)


def kernel(user_emb, item_emb, edge_index, edge_weight):
    ego0 = jnp.concatenate([user_emb, item_emb], axis=0)
    ego0 = ego0.reshape(N_NODES, 2, HALF).transpose(1, 0, 2)
    ego0 = jnp.pad(ego0, ((0, 0), (0, N_PAD - N_NODES), (0, 0)))
    ego0 = ego0.reshape(2 * N_PAD, HALF)
    pad = E_PAD - N_EDGES
    src = jnp.concatenate([edge_index[0], jnp.zeros((pad,), jnp.int32)])
    dst = jnp.concatenate([edge_index[1], jnp.zeros((pad,), jnp.int32)])
    dst = dst.reshape(N_TILES, BLOCKS * CHUNKS_PER_BLOCK, CHUNK)
    w = jnp.concatenate([edge_weight, jnp.zeros((pad,), jnp.float32)])
    out_sum, _ = _sc_kernel(ego0, src, dst, w)
    out = out_sum.reshape(2, N_PAD, HALF)[:, :N_NODES].transpose(
        1, 0, 2).reshape(N_NODES, EMB)
    return out[:N_USER], out[N_USER:]


# speedup vs baseline: 9.5328x; 1.0013x over previous
import functools

import jax
import jax.numpy as jnp
from jax import lax
from jax.experimental import pallas as pl
from jax.experimental.pallas import tpu as pltpu
from jax.experimental.pallas import tpu_sc as plsc

N_USER = 20000
N_ITEM = 30000
N_NODES = N_USER + N_ITEM
N_PAD = 50048
EMB = 64
HALF = 32
N_EDGES = 800000
N_TILES = 16
ROWS_PT = N_PAD // N_TILES
CHUNK = 128
NBUF = 4
CHUNKS_PER_BLOCK = 28
BLOCK_E = CHUNK * CHUNKS_PER_BLOCK
BLOCKS = 14
EDGES_PT = BLOCK_E * BLOCKS
E_PAD = EDGES_PT * N_TILES
WB_FULL = ROWS_PT // CHUNK
WB_TAIL = ROWS_PT - WB_FULL * CHUNK
LANES = 16


def _sc_body(ego0, srcr, dstr, wr, out_sum, ego_scr, spmem,
             src_blk, w_blk, dst_blk, gb0, gb1, gb2, gb3,
             gs0, gs1, gs2, gs3, ss0, ss1, ss2, ss3):
    c = lax.axis_index("c")
    s = lax.axis_index("s")
    col_off = c * N_PAD
    offv = jnp.broadcast_to(col_off, (LANES,)).astype(jnp.int32)
    third = jnp.float32(1.0 / 3.0)
    zv = jnp.zeros((LANES,), jnp.float32)
    gbufs = [gb0, gb1, gb2, gb3]
    gsems = [gs0, gs1, gs2, gs3]
    ssems = [ss0, ss1, ss2, ss3]
    ebuf, abuf = gb0, gb1

    def layer(src_ref, first, last):
      with jax.named_scope("phaseZ"):
        def zfill(i, _):
            for h in range(2):
                ebuf[i, pl.ds(h * LANES, LANES)] = zv
            return 0

        lax.fori_loop(0, CHUNK, zfill, 0)

        def zcopy(k, _):
            pltpu.sync_copy(ebuf,
                            spmem.at[pl.ds(s * ROWS_PT + k * CHUNK, CHUNK)])
            return 0

        lax.fori_loop(0, WB_FULL, zcopy, 0)
        pltpu.sync_copy(
            ebuf.at[pl.ds(0, WB_TAIL)],
            spmem.at[pl.ds(s * ROWS_PT + WB_FULL * CHUNK, WB_TAIL)])
        plsc.subcore_barrier()

      with jax.named_scope("phaseS"):
        def gidx(j):
            return src_blk.at[pl.ds(j * CHUNK, CHUNK)]

        def block_body(b, _):
            eoff = s * EDGES_PT + b * BLOCK_E
            pltpu.sync_copy(srcr.at[pl.ds(eoff, BLOCK_E)], src_blk)
            pltpu.sync_copy(wr.at[pl.ds(eoff, BLOCK_E)], w_blk)
            pltpu.sync_copy(
                dstr.at[s, pl.ds(b * CHUNKS_PER_BLOCK, CHUNKS_PER_BLOCK)],
                dst_blk)

            def add_off(i, _):
                src_blk[pl.ds(i * LANES, LANES)] = (
                    src_blk[pl.ds(i * LANES, LANES)] + offv)
                return 0

            lax.fori_loop(0, BLOCK_E // LANES, add_off, 0)

            for p in range(NBUF):
                pltpu.async_copy(src_ref.at[gidx(p)], gbufs[p], gsems[p])

            def quad_body(jq, _):
                for p in range(NBUF):
                    j = NBUF * jq + p
                    pltpu.make_async_copy(
                        src_ref.at[gidx(j)], gbufs[p], gsems[p]).wait()
                    for g in range(CHUNK // LANES):
                        wvec = w_blk[pl.ds(j * CHUNK + g * LANES, LANES)]
                        for i in range(LANES):
                            row = g * LANES + i
                            wv = jnp.broadcast_to(wvec[i], (LANES,))
                            for h in range(2):
                                gbufs[p][row, pl.ds(h * LANES, LANES)] = (
                                    gbufs[p][row, pl.ds(h * LANES, LANES)]
                                    * wv)
                    pltpu.async_copy(gbufs[p], spmem.at[dst_blk.at[j]],
                                     ssems[p], add=True)

                    @pl.when(j + NBUF < CHUNKS_PER_BLOCK)
                    def _():
                        pltpu.make_async_copy(
                            gbufs[p], spmem.at[dst_blk.at[j]],
                            ssems[p]).wait()
                        pltpu.async_copy(src_ref.at[gidx(j + NBUF)],
                                         gbufs[p], gsems[p])
                return 0

            lax.fori_loop(0, CHUNKS_PER_BLOCK // NBUF, quad_body, 0)
            for p in range(NBUF):
                j = CHUNKS_PER_BLOCK - NBUF + p
                pltpu.make_async_copy(
                    gbufs[p], spmem.at[dst_blk.at[j]], ssems[p]).wait()
            return 0

        lax.fori_loop(0, BLOCKS, block_body, 0)
        plsc.subcore_barrier()

      with jax.named_scope("phaseW"):
        def wb_chunk(r0, rows):
            pltpu.sync_copy(spmem.at[pl.ds(r0, rows)],
                            ebuf.at[pl.ds(0, rows)])
            if not last:
                pltpu.sync_copy(ebuf.at[pl.ds(0, rows)],
                                ego_scr.at[pl.ds(col_off + r0, rows)])
            if not first:
                pltpu.sync_copy(out_sum.at[pl.ds(col_off + r0, rows)],
                                abuf.at[pl.ds(0, rows)])

            def wb_row(i, _):
                for h in range(2):
                    v = ebuf[i, pl.ds(h * LANES, LANES)] * third
                    if first:
                        abuf[i, pl.ds(h * LANES, LANES)] = v
                    else:
                        abuf[i, pl.ds(h * LANES, LANES)] = (
                            abuf[i, pl.ds(h * LANES, LANES)] + v)
                return 0

            lax.fori_loop(0, rows, wb_row, 0)
            pltpu.sync_copy(abuf.at[pl.ds(0, rows)],
                            out_sum.at[pl.ds(col_off + r0, rows)])

        def wb_body(k, _):
            wb_chunk(s * ROWS_PT + k * CHUNK, CHUNK)
            return 0

        lax.fori_loop(0, WB_FULL, wb_body, 0)
        wb_chunk(s * ROWS_PT + WB_FULL * CHUNK, WB_TAIL)

    layer(ego0, first=True, last=False)
    plsc.subcore_barrier()
    layer(ego_scr, first=False, last=False)
    plsc.subcore_barrier()
    layer(ego_scr, first=False, last=True)


_sc_kernel = functools.partial(
    pl.kernel,
    out_type=(
        jax.ShapeDtypeStruct((2 * N_PAD, HALF), jnp.float32),
        jax.ShapeDtypeStruct((2 * N_PAD, HALF), jnp.float32),
    ),
    mesh=plsc.VectorSubcoreMesh(core_axis_name="c", subcore_axis_name="s"),
    compiler_params=pltpu.CompilerParams(use_tc_tiling_on_sc=False),
    scratch_types=[
        pltpu.VMEM_SHARED((N_PAD, HALF), jnp.float32),
        pltpu.VMEM((BLOCK_E,), jnp.int32),
        pltpu.VMEM((BLOCK_E,), jnp.float32),
        pltpu.VMEM((CHUNKS_PER_BLOCK, CHUNK), jnp.int32),
        pltpu.VMEM((CHUNK, HALF), jnp.float32),
        pltpu.VMEM((CHUNK, HALF), jnp.float32),
        pltpu.VMEM((CHUNK, HALF), jnp.float32),
        pltpu.VMEM((CHUNK, HALF), jnp.float32),
        pltpu.SemaphoreType.DMA,
        pltpu.SemaphoreType.DMA,
        pltpu.SemaphoreType.DMA,
        pltpu.SemaphoreType.DMA,
        pltpu.SemaphoreType.DMA,
        pltpu.SemaphoreType.DMA,
        pltpu.SemaphoreType.DMA,
        pltpu.SemaphoreType.DMA,
    ],
)(_sc_body)


def kernel(user_emb, item_emb, edge_index, edge_weight):
    ego0 = jnp.concatenate([user_emb, item_emb], axis=0)
    ego0 = ego0.reshape(N_NODES, 2, HALF).transpose(1, 0, 2)
    ego0 = jnp.pad(ego0, ((0, 0), (0, N_PAD - N_NODES), (0, 0)))
    ego0 = ego0.reshape(2 * N_PAD, HALF)
    pad = E_PAD - N_EDGES
    src = jnp.concatenate([edge_index[0], jnp.zeros((pad,), jnp.int32)])
    dst = jnp.concatenate([edge_index[1], jnp.zeros((pad,), jnp.int32)])
    dst = dst.reshape(N_TILES, BLOCKS * CHUNKS_PER_BLOCK, CHUNK)
    w = jnp.concatenate([edge_weight, jnp.zeros((pad,), jnp.float32)])
    out_sum, _ = _sc_kernel(ego0, src, dst, w)
    out = out_sum.reshape(2, N_PAD, HALF)[:, :N_NODES].transpose(
        1, 0, 2).reshape(N_NODES, EMB)
    return out[:N_USER], out[N_USER:]
